# Initial kernel scaffold; baseline (speedup 1.0000x reference)
import jax
import jax.numpy as jnp
from jax.experimental import pallas as pl


def kernel(feat, edge_index, W_Q, b_Q, W_K, b_K, W_V, b_V):
    raise NotImplementedError("write your pallas kernel here")


# speedup vs baseline: 10.4609x; 10.4609x over previous
import functools

import jax
import jax.numpy as jnp
from jax import lax
from jax.experimental import pallas as pl
from jax.experimental.pallas import tpu as pltpu
from jax.experimental.pallas import tpu_sc as plsc

N = 10000
E = 320000
D = 128
H = 8
HD = 16

NC = 2
NS = 16
L = 16
NW = NC * NS
EPW = E // NW
BATCH = 80
NB = EPW // BATCH
G = BATCH // L
RPT = N // NS
CROWS = 80
NCHUNK = N // CROWS
CPT = 8

_RB = 1000


def _proj_body(feat, wq, bq, wk, bk, wv, bv, q, k, v):
    f = feat[...]
    dn = (((1,), (1,)), ((), ()))
    q[...] = lax.dot_general(f, wq[...], dn, preferred_element_type=jnp.float32) + bq[...]
    k[...] = lax.dot_general(f, wk[...], dn, preferred_element_type=jnp.float32) + bk[...]
    v[...] = lax.dot_general(f, wv[...], dn, preferred_element_type=jnp.float32) + bv[...]


def _project(feat, W_Q, b_Q, W_K, b_K, W_V, b_V):
    grid = (N // _RB,)
    row_spec = pl.BlockSpec((_RB, D), lambda i: (i, 0))
    w_spec = pl.BlockSpec((D, D), lambda i: (0, 0))
    b_spec = pl.BlockSpec((1, D), lambda i: (0, 0))
    return pl.pallas_call(
        _proj_body,
        grid=grid,
        in_specs=[row_spec, w_spec, b_spec, w_spec, b_spec, w_spec, b_spec],
        out_specs=[row_spec, row_spec, row_spec],
        out_shape=[jax.ShapeDtypeStruct((N, D), jnp.float32)] * 3,
    )(feat, W_Q, b_Q.reshape(1, D), W_K, b_K.reshape(1, D), W_V, b_V.reshape(1, D))


def _score_body(src, dst, k_hbm, q_hbm, score, tmax,
                sidx, didx, kbuf, qbuf, sbuf, mbuf, sem0, sem1):
    cid = lax.axis_index("c")
    sid = lax.axis_index("s")
    wid = sid * NC + cid
    base = wid * EPW
    lanes = jnp.arange(L, dtype=jnp.int32)

    def batch_body(b, maxacc):
        off = base + b * BATCH
        pltpu.sync_copy(src.at[pl.ds(off, BATCH)], sidx)
        pltpu.sync_copy(dst.at[pl.ds(off, BATCH)], didx)
        cp0 = pltpu.async_copy(k_hbm.at[sidx], kbuf, sem0)
        cp1 = pltpu.async_copy(q_hbm.at[didx], qbuf, sem1)
        cp0.wait()
        cp1.wait()

        def group_body(g, macc):
            rows = g * L + lanes
            for h in range(H):
                acc = jnp.zeros((L,), jnp.float32)
                for d in range(HD):
                    cc = jnp.full((L,), h * HD + d, jnp.int32)
                    kc = plsc.load_gather(kbuf, [rows, cc])
                    qc = plsc.load_gather(qbuf, [rows, cc])
                    acc = acc + kc * qc
                acc = acc * 0.25
                plsc.store_scatter(sbuf, [rows, jnp.full((L,), h, jnp.int32)], acc)
                macc = jnp.maximum(macc, acc)
            return macc

        maxacc = lax.fori_loop(0, G, group_body, maxacc)
        pltpu.sync_copy(sbuf, score.at[pl.ds(off, BATCH)])
        return maxacc

    maxacc = lax.fori_loop(0, NB, batch_body,
                           jnp.full((L,), -jnp.inf, jnp.float32))
    mbuf[...] = maxacc
    pltpu.sync_copy(mbuf, tmax.at[wid])


def _scores(src, dst, k, q):
    mesh = plsc.VectorSubcoreMesh(core_axis_name="c", subcore_axis_name="s",
                                  num_cores=NC, num_subcores=NS)
    f = pl.kernel(
        _score_body,
        out_type=(jax.ShapeDtypeStruct((E, H), jnp.float32),
                  jax.ShapeDtypeStruct((NW, L), jnp.float32)),
        mesh=mesh,
        compiler_params=pltpu.CompilerParams(needs_layout_passes=False),
        scratch_types=[
            pltpu.VMEM((BATCH,), jnp.int32),
            pltpu.VMEM((BATCH,), jnp.int32),
            pltpu.VMEM((BATCH, D), jnp.float32),
            pltpu.VMEM((BATCH, D), jnp.float32),
            pltpu.VMEM((BATCH, H), jnp.float32),
            pltpu.VMEM((L,), jnp.float32),
            pltpu.SemaphoreType.DMA,
            pltpu.SemaphoreType.DMA,
        ],
    )
    return f(src, dst, k, q)


def _agg_body(src, dst, v_hbm, score, tmax, zm, hpart,
              sidx, didx, vbuf, msg, sbuf, tmv, h_sp, sem0):
    cid = lax.axis_index("c")
    sid = lax.axis_index("s")
    wid = sid * NC + cid
    base = wid * EPW
    lanes = jnp.arange(L, dtype=jnp.int32)

    pltpu.sync_copy(zm, msg)

    def zero_chunk(i, carry):
        c = jnp.minimum(sid * CPT + i, NCHUNK - 1)
        pltpu.sync_copy(msg, h_sp.at[pl.ds(c * CROWS, CROWS)])
        return carry

    lax.fori_loop(0, CPT, zero_chunk, 0)

    pltpu.sync_copy(tmax, tmv)
    mv = jnp.full((L,), -jnp.inf, jnp.float32)
    for i in range(NW):
        mv = jnp.maximum(mv, tmv[i])
    M = jnp.max(mv)

    plsc.subcore_barrier()

    def batch_body(b, carry):
        off = base + b * BATCH
        pltpu.sync_copy(src.at[pl.ds(off, BATCH)], sidx)
        pltpu.sync_copy(dst.at[pl.ds(off, BATCH)], didx)
        cp = pltpu.async_copy(v_hbm.at[sidx], vbuf, sem0)
        pltpu.sync_copy(score.at[pl.ds(off, BATCH)], sbuf)
        cp.wait()

        def group_body(g, c2):
            rows = g * L + lanes
            for h in range(H):
                hc = jnp.full((L,), h, jnp.int32)
                sv = plsc.load_gather(sbuf, [rows, hc])
                ev = jnp.exp(sv - M)
                for d in range(HD):
                    cc = jnp.full((L,), h * HD + d, jnp.int32)
                    col = plsc.load_gather(vbuf, [rows, cc])
                    plsc.store_scatter(msg, [rows, cc], col * ev)
            return c2

        lax.fori_loop(0, G, group_body, 0)
        pltpu.sync_copy(msg, h_sp.at[didx], add=True)
        return carry

    lax.fori_loop(0, NB, batch_body, 0)

    plsc.subcore_barrier()

    def out_chunk(i, carry):
        c = jnp.minimum(sid * CPT + i, NCHUNK - 1)
        pltpu.sync_copy(h_sp.at[pl.ds(c * CROWS, CROWS)], vbuf)
        pltpu.sync_copy(vbuf, hpart.at[cid, c])
        return carry

    lax.fori_loop(0, CPT, out_chunk, 0)


def _aggregate(src, dst, v, score, tmax):
    mesh = plsc.VectorSubcoreMesh(core_axis_name="c", subcore_axis_name="s",
                                  num_cores=NC, num_subcores=NS)
    zm = jnp.zeros((BATCH, D), jnp.float32)
    f = pl.kernel(
        _agg_body,
        out_type=jax.ShapeDtypeStruct((NC, NCHUNK, CROWS, D), jnp.float32),
        mesh=mesh,
        compiler_params=pltpu.CompilerParams(needs_layout_passes=False),
        scratch_types=[
            pltpu.VMEM((BATCH,), jnp.int32),
            pltpu.VMEM((BATCH,), jnp.int32),
            pltpu.VMEM((BATCH, D), jnp.float32),
            pltpu.VMEM((BATCH, D), jnp.float32),
            pltpu.VMEM((BATCH, H), jnp.float32),
            pltpu.VMEM((NW, L), jnp.float32),
            pltpu.VMEM_SHARED((N, D), jnp.float32),
            pltpu.SemaphoreType.DMA,
        ],
    )
    return f(src, dst, v, score, tmax, zm)


def _den_body(dst, score, tmax, zd, dpart, didx, sbuf, tmv, dtile):
    cid = lax.axis_index("c")
    sid = lax.axis_index("s")
    wid = sid * NC + cid
    base = wid * EPW
    lanes = jnp.arange(L, dtype=jnp.int32)

    pltpu.sync_copy(zd, dtile)

    pltpu.sync_copy(tmax, tmv)
    mv = jnp.full((L,), -jnp.inf, jnp.float32)
    for i in range(NW):
        mv = jnp.maximum(mv, tmv[i])
    M = jnp.max(mv)

    def batch_body(b, carry):
        off = base + b * BATCH
        pltpu.sync_copy(dst.at[pl.ds(off, BATCH)], didx)
        pltpu.sync_copy(score.at[pl.ds(off, BATCH)], sbuf)

        def group_body(g, c2):
            rows = g * L + lanes
            dd = didx[pl.ds(g * L, L)] * H
            for h in range(H):
                hc = jnp.full((L,), h, jnp.int32)
                sv = plsc.load_gather(sbuf, [rows, hc])
                ev = jnp.exp(sv - M)
                plsc.addupdate_scatter(dtile, [dd + h], ev)
            return c2

        lax.fori_loop(0, G, group_body, 0)
        return carry

    lax.fori_loop(0, NB, batch_body, 0)
    pltpu.sync_copy(dtile, dpart.at[cid, sid])


def _denominators(dst, score, tmax):
    mesh = plsc.VectorSubcoreMesh(core_axis_name="c", subcore_axis_name="s",
                                  num_cores=NC, num_subcores=NS)
    zd = jnp.zeros((N * H,), jnp.float32)
    f = pl.kernel(
        _den_body,
        out_type=jax.ShapeDtypeStruct((NC, NS, N * H), jnp.float32),
        mesh=mesh,
        compiler_params=pltpu.CompilerParams(needs_layout_passes=False),
        scratch_types=[
            pltpu.VMEM((BATCH,), jnp.int32),
            pltpu.VMEM((BATCH, H), jnp.float32),
            pltpu.VMEM((NW, L), jnp.float32),
            pltpu.VMEM((N * H,), jnp.float32),
        ],
    )
    return f(dst, score, tmax, zd)


def _final_body(hp0, hp1, dpp, out):
    hsum = hp0[...] + hp1[...]
    d8 = jnp.sum(dpp[...], axis=0)
    safe = jnp.where(d8 == 0.0, 1.0, d8)
    r = 1.0 / safe
    rowi = lax.broadcasted_iota(jnp.int32, (H, D), 0)
    coli = lax.broadcasted_iota(jnp.int32, (H, D), 1)
    em = jnp.where(rowi == coli // HD, 1.0, 0.0).astype(jnp.float32)
    dex = lax.dot_general(r, em, (((1,), (0,)), ((), ())),
                          preferred_element_type=jnp.float32)
    out[...] = hsum * dex


def _finalize(hp0, hp1, dpp):
    grid = (N // _RB,)
    h_spec = pl.BlockSpec((_RB, D), lambda i: (i, 0))
    d_spec = pl.BlockSpec((NW, _RB, H), lambda i: (0, i, 0))
    return pl.pallas_call(
        _final_body,
        grid=grid,
        in_specs=[h_spec, h_spec, d_spec],
        out_specs=h_spec,
        out_shape=jax.ShapeDtypeStruct((N, D), jnp.float32),
    )(hp0, hp1, dpp)


def kernel(feat, edge_index, W_Q, b_Q, W_K, b_K, W_V, b_V):
    src = edge_index[0]
    dst = edge_index[1]
    q, k, v = _project(feat, W_Q, b_Q, W_K, b_K, W_V, b_V)
    score, tmax = _scores(src, dst, k, q)
    hp = _aggregate(src, dst, v, score, tmax)
    dp = _denominators(dst, score, tmax)
    hp = hp.reshape(NC, N, D)
    dpp = dp.reshape(NW, N, H)
    return _finalize(hp[0], hp[1], dpp)


# speedup vs baseline: 22.2939x; 2.1312x over previous
import functools

import jax
import jax.numpy as jnp
from jax import lax
from jax.experimental import pallas as pl
from jax.experimental.pallas import tpu as pltpu
from jax.experimental.pallas import tpu_sc as plsc

N = 10000
E = 320000
D = 128
H = 8
HD = 16

NC = 2
NS = 16
L = 16
NW = NC * NS
EPW = E // NW
BATCH = 80
NB = EPW // BATCH
G = BATCH // L
RPT = N // NS
CROWS = 80
NCHUNK = N // CROWS
CPT = 8

_RB = 1000


def _proj_body(feat, wq, bq, wk, bk, wv, bv, q, k, v):
    f = feat[...]
    dn = (((1,), (1,)), ((), ()))
    q[...] = lax.dot_general(f, wq[...], dn, preferred_element_type=jnp.float32) + bq[...]
    k[...] = lax.dot_general(f, wk[...], dn, preferred_element_type=jnp.float32) + bk[...]
    v[...] = lax.dot_general(f, wv[...], dn, preferred_element_type=jnp.float32) + bv[...]


def _project(feat, W_Q, b_Q, W_K, b_K, W_V, b_V):
    grid = (N // _RB,)
    row_spec = pl.BlockSpec((_RB, D), lambda i: (i, 0))
    w_spec = pl.BlockSpec((D, D), lambda i: (0, 0))
    b_spec = pl.BlockSpec((1, D), lambda i: (0, 0))
    return pl.pallas_call(
        _proj_body,
        grid=grid,
        in_specs=[row_spec, w_spec, b_spec, w_spec, b_spec, w_spec, b_spec],
        out_specs=[row_spec, row_spec, row_spec],
        out_shape=[jax.ShapeDtypeStruct((N, D), jnp.float32)] * 3,
    )(feat, W_Q, b_Q.reshape(1, D), W_K, b_K.reshape(1, D), W_V, b_V.reshape(1, D))


def _score_body(src, dst, k_hbm, q_hbm, score, tmax,
                sidx, didx, kbuf, qbuf, sbuf, mbuf, sem0, sem1):
    cid = lax.axis_index("c")
    sid = lax.axis_index("s")
    wid = sid * NC + cid
    base = wid * EPW
    lanes = jnp.arange(L, dtype=jnp.int32)

    def batch_body(b, maxacc):
        off = base + b * BATCH
        pltpu.sync_copy(src.at[pl.ds(off, BATCH)], sidx)
        pltpu.sync_copy(dst.at[pl.ds(off, BATCH)], didx)
        cp0 = pltpu.async_copy(k_hbm.at[sidx], kbuf, sem0)
        cp1 = pltpu.async_copy(q_hbm.at[didx], qbuf, sem1)
        cp0.wait()
        cp1.wait()

        idx15 = jnp.full((L,), 15, jnp.int32)
        neg = jnp.full((L,), -jnp.inf, jnp.float32)

        def group_body(g, macc):
            for j in range(L):
                row = g * L + j
                r = neg
                for h in range(H):
                    kc = kbuf[row, pl.ds(h * HD, HD)]
                    qc = qbuf[row, pl.ds(h * HD, HD)]
                    s = jnp.cumsum(kc * qc)
                    tot = lax.gather(
                        s, idx15[:, None],
                        lax.GatherDimensionNumbers(
                            offset_dims=(), collapsed_slice_dims=(0,),
                            start_index_map=(0,)),
                        (1,), mode=lax.GatherScatterMode.PROMISE_IN_BOUNDS)
                    r = jnp.where(lanes == h, tot, r)
                r = r * 0.25
                plsc.store_scatter(sbuf, [jnp.full((L,), row, jnp.int32), lanes],
                                   r, mask=lanes < H)
                macc = jnp.maximum(macc, jnp.where(lanes < H, r, neg))
            return macc

        maxacc = lax.fori_loop(0, G, group_body, maxacc)
        pltpu.sync_copy(sbuf, score.at[pl.ds(off, BATCH)])
        return maxacc

    maxacc = lax.fori_loop(0, NB, batch_body,
                           jnp.full((L,), -jnp.inf, jnp.float32))
    mbuf[...] = maxacc
    pltpu.sync_copy(mbuf, tmax.at[wid])


def _scores(src, dst, k, q):
    mesh = plsc.VectorSubcoreMesh(core_axis_name="c", subcore_axis_name="s",
                                  num_cores=NC, num_subcores=NS)
    f = pl.kernel(
        _score_body,
        out_type=(jax.ShapeDtypeStruct((E, H), jnp.float32),
                  jax.ShapeDtypeStruct((NW, L), jnp.float32)),
        mesh=mesh,
        compiler_params=pltpu.CompilerParams(needs_layout_passes=False),
        scratch_types=[
            pltpu.VMEM((BATCH,), jnp.int32),
            pltpu.VMEM((BATCH,), jnp.int32),
            pltpu.VMEM((BATCH, D), jnp.float32),
            pltpu.VMEM((BATCH, D), jnp.float32),
            pltpu.VMEM((BATCH, H), jnp.float32),
            pltpu.VMEM((L,), jnp.float32),
            pltpu.SemaphoreType.DMA,
            pltpu.SemaphoreType.DMA,
        ],
    )
    return f(src, dst, k, q)


def _agg_body(src, dst, v_hbm, score, tmax, zm, hpart,
              sidx, didx, vbuf, msg, sbuf, tmv, epad, h_sp, sem0):
    cid = lax.axis_index("c")
    sid = lax.axis_index("s")
    wid = sid * NC + cid
    base = wid * EPW
    lanes = jnp.arange(L, dtype=jnp.int32)

    pltpu.sync_copy(zm, msg)

    def zero_chunk(i, carry):
        c = jnp.minimum(sid * CPT + i, NCHUNK - 1)
        pltpu.sync_copy(msg, h_sp.at[pl.ds(c * CROWS, CROWS)])
        return carry

    lax.fori_loop(0, CPT, zero_chunk, 0)

    pltpu.sync_copy(tmax, tmv)
    mv = jnp.full((L,), -jnp.inf, jnp.float32)
    for i in range(NW):
        mv = jnp.maximum(mv, tmv[i])
    M = jnp.max(mv)

    plsc.subcore_barrier()

    def batch_body(b, carry):
        off = base + b * BATCH
        pltpu.sync_copy(src.at[pl.ds(off, BATCH)], sidx)
        pltpu.sync_copy(dst.at[pl.ds(off, BATCH)], didx)
        cp = pltpu.async_copy(v_hbm.at[sidx], vbuf, sem0)
        pltpu.sync_copy(score.at[pl.ds(off, BATCH)], sbuf)
        cp.wait()

        def group_body(g, c2):
            rows = g * L + lanes
            for h in range(H):
                hc = jnp.full((L,), h, jnp.int32)
                sv = plsc.load_gather(sbuf, [rows, hc])
                ev = jnp.exp(sv - M)
                plsc.store_scatter(epad, [rows * 9 + h], ev)
            for j in range(L):
                row = g * L + j
                for h in range(H):
                    se = plsc.load_gather(
                        epad, [jnp.full((L,), row * 9 + h, jnp.int32)])
                    c0 = h * HD
                    msg[row, pl.ds(c0, HD)] = vbuf[row, pl.ds(c0, HD)] * se
            return c2

        lax.fori_loop(0, G, group_body, 0)
        pltpu.sync_copy(msg, h_sp.at[didx], add=True)
        return carry

    lax.fori_loop(0, NB, batch_body, 0)

    plsc.subcore_barrier()

    def out_chunk(i, carry):
        c = jnp.minimum(sid * CPT + i, NCHUNK - 1)
        pltpu.sync_copy(h_sp.at[pl.ds(c * CROWS, CROWS)], vbuf)
        pltpu.sync_copy(vbuf, hpart.at[cid, c])
        return carry

    lax.fori_loop(0, CPT, out_chunk, 0)


def _aggregate(src, dst, v, score, tmax):
    mesh = plsc.VectorSubcoreMesh(core_axis_name="c", subcore_axis_name="s",
                                  num_cores=NC, num_subcores=NS)
    zm = jnp.zeros((BATCH, D), jnp.float32)
    f = pl.kernel(
        _agg_body,
        out_type=jax.ShapeDtypeStruct((NC, NCHUNK, CROWS, D), jnp.float32),
        mesh=mesh,
        compiler_params=pltpu.CompilerParams(needs_layout_passes=False),
        scratch_types=[
            pltpu.VMEM((BATCH,), jnp.int32),
            pltpu.VMEM((BATCH,), jnp.int32),
            pltpu.VMEM((BATCH, D), jnp.float32),
            pltpu.VMEM((BATCH, D), jnp.float32),
            pltpu.VMEM((BATCH, H), jnp.float32),
            pltpu.VMEM((NW, L), jnp.float32),
            pltpu.VMEM((BATCH * 9,), jnp.float32),
            pltpu.VMEM_SHARED((N, D), jnp.float32),
            pltpu.SemaphoreType.DMA,
        ],
    )
    return f(src, dst, v, score, tmax, zm)


def _den_body(dst, score, tmax, zd, dpart, didx, sbuf, tmv, dtile):
    cid = lax.axis_index("c")
    sid = lax.axis_index("s")
    wid = sid * NC + cid
    base = wid * EPW
    lanes = jnp.arange(L, dtype=jnp.int32)

    pltpu.sync_copy(zd, dtile)

    pltpu.sync_copy(tmax, tmv)
    mv = jnp.full((L,), -jnp.inf, jnp.float32)
    for i in range(NW):
        mv = jnp.maximum(mv, tmv[i])
    M = jnp.max(mv)

    def batch_body(b, carry):
        off = base + b * BATCH
        pltpu.sync_copy(dst.at[pl.ds(off, BATCH)], didx)
        pltpu.sync_copy(score.at[pl.ds(off, BATCH)], sbuf)

        def group_body(g, c2):
            rows = g * L + lanes
            dd = didx[pl.ds(g * L, L)] * H
            for h in range(H):
                hc = jnp.full((L,), h, jnp.int32)
                sv = plsc.load_gather(sbuf, [rows, hc])
                ev = jnp.exp(sv - M)
                plsc.addupdate_scatter(dtile, [dd + h], ev)
            return c2

        lax.fori_loop(0, G, group_body, 0)
        return carry

    lax.fori_loop(0, NB, batch_body, 0)
    pltpu.sync_copy(dtile, dpart.at[cid, sid])


def _denominators(dst, score, tmax):
    mesh = plsc.VectorSubcoreMesh(core_axis_name="c", subcore_axis_name="s",
                                  num_cores=NC, num_subcores=NS)
    zd = jnp.zeros((N * H,), jnp.float32)
    f = pl.kernel(
        _den_body,
        out_type=jax.ShapeDtypeStruct((NC, NS, N * H), jnp.float32),
        mesh=mesh,
        compiler_params=pltpu.CompilerParams(needs_layout_passes=False),
        scratch_types=[
            pltpu.VMEM((BATCH,), jnp.int32),
            pltpu.VMEM((BATCH, H), jnp.float32),
            pltpu.VMEM((NW, L), jnp.float32),
            pltpu.VMEM((N * H,), jnp.float32),
        ],
    )
    return f(dst, score, tmax, zd)


def _final_body(hp0, hp1, dpp, out):
    hsum = hp0[...] + hp1[...]
    d8 = jnp.sum(dpp[...], axis=0)
    safe = jnp.where(d8 == 0.0, 1.0, d8)
    r = 1.0 / safe
    rowi = lax.broadcasted_iota(jnp.int32, (H, D), 0)
    coli = lax.broadcasted_iota(jnp.int32, (H, D), 1)
    em = jnp.where(rowi == coli // HD, 1.0, 0.0).astype(jnp.float32)
    dex = lax.dot_general(r, em, (((1,), (0,)), ((), ())),
                          preferred_element_type=jnp.float32)
    out[...] = hsum * dex


def _finalize(hp0, hp1, dpp):
    grid = (N // _RB,)
    h_spec = pl.BlockSpec((_RB, D), lambda i: (i, 0))
    d_spec = pl.BlockSpec((NW, _RB, H), lambda i: (0, i, 0))
    return pl.pallas_call(
        _final_body,
        grid=grid,
        in_specs=[h_spec, h_spec, d_spec],
        out_specs=h_spec,
        out_shape=jax.ShapeDtypeStruct((N, D), jnp.float32),
    )(hp0, hp1, dpp)


def kernel(feat, edge_index, W_Q, b_Q, W_K, b_K, W_V, b_V):
    src = edge_index[0]
    dst = edge_index[1]
    q, k, v = _project(feat, W_Q, b_Q, W_K, b_K, W_V, b_V)
    score, tmax = _scores(src, dst, k, q)
    hp = _aggregate(src, dst, v, score, tmax)
    dp = _denominators(dst, score, tmax)
    hp = hp.reshape(NC, N, D)
    dpp = dp.reshape(NW, N, H)
    return _finalize(hp[0], hp[1], dpp)


# speedup vs baseline: 24.7028x; 1.1080x over previous
import functools

import jax
import jax.numpy as jnp
from jax import lax
from jax.experimental import pallas as pl
from jax.experimental.pallas import tpu as pltpu
from jax.experimental.pallas import tpu_sc as plsc

N = 10000
E = 320000
D = 128
H = 8
HD = 16

NC = 2
NS = 16
L = 16
NW = NC * NS
EPW = E // NW
BATCH = 80
NB = EPW // BATCH
G = BATCH // L
RPT = N // NS
CROWS = 80
NCHUNK = N // CROWS
CPT = 8

DBATCH = 80

_RB = 1000


def _proj_body(feat, wq, bq, wk, bk, wv, bv, q, k, v):
    f = feat[...]
    dn = (((1,), (1,)), ((), ()))
    q[...] = lax.dot_general(f, wq[...], dn, preferred_element_type=jnp.float32) + bq[...]
    k[...] = lax.dot_general(f, wk[...], dn, preferred_element_type=jnp.float32) + bk[...]
    v[...] = lax.dot_general(f, wv[...], dn, preferred_element_type=jnp.float32) + bv[...]


def _project(feat, W_Q, b_Q, W_K, b_K, W_V, b_V):
    grid = (N // _RB,)
    row_spec = pl.BlockSpec((_RB, D), lambda i: (i, 0))
    w_spec = pl.BlockSpec((D, D), lambda i: (0, 0))
    b_spec = pl.BlockSpec((1, D), lambda i: (0, 0))
    return pl.pallas_call(
        _proj_body,
        grid=grid,
        in_specs=[row_spec, w_spec, b_spec, w_spec, b_spec, w_spec, b_spec],
        out_specs=[row_spec, row_spec, row_spec],
        out_shape=[jax.ShapeDtypeStruct((N, D), jnp.float32)] * 3,
    )(feat, W_Q, b_Q.reshape(1, D), W_K, b_K.reshape(1, D), W_V, b_V.reshape(1, D))


def _score_body(src, dst, k_hbm, q_hbm, score, tmax,
                sall, dall, kbuf0, qbuf0, kbuf1, qbuf1, sbuf, mbuf,
                sem0, sem1):
    cid = lax.axis_index("c")
    sid = lax.axis_index("s")
    wid = sid * NC + cid
    base = wid * EPW
    lanes = jnp.arange(L, dtype=jnp.int32)

    pltpu.sync_copy(src.at[pl.ds(base, EPW)], sall)
    pltpu.sync_copy(dst.at[pl.ds(base, EPW)], dall)

    def start(b, buf_k, buf_q, sem):
        i0 = b * BATCH
        ck = pltpu.async_copy(k_hbm.at[sall.at[pl.ds(i0, BATCH)]], buf_k, sem)
        cq = pltpu.async_copy(q_hbm.at[dall.at[pl.ds(i0, BATCH)]], buf_q, sem)
        return ck, cq

    def wait(b, buf_k, buf_q, sem):
        i0 = b * BATCH
        pltpu.make_async_copy(k_hbm.at[sall.at[pl.ds(i0, BATCH)]], buf_k, sem).wait()
        pltpu.make_async_copy(q_hbm.at[dall.at[pl.ds(i0, BATCH)]], buf_q, sem).wait()

    idx15 = jnp.full((L,), 15, jnp.int32)
    neg = jnp.full((L,), -jnp.inf, jnp.float32)

    def compute(b, buf_k, buf_q, maxacc):
        def group_body(g, macc):
            for j in range(L):
                row = g * L + j
                r = neg
                for h in range(H):
                    kc = buf_k[row, pl.ds(h * HD, HD)]
                    qc = buf_q[row, pl.ds(h * HD, HD)]
                    s = jnp.cumsum(kc * qc)
                    tot = lax.gather(
                        s, idx15[:, None],
                        lax.GatherDimensionNumbers(
                            offset_dims=(), collapsed_slice_dims=(0,),
                            start_index_map=(0,)),
                        (1,), mode=lax.GatherScatterMode.PROMISE_IN_BOUNDS)
                    r = jnp.where(lanes == h, tot, r)
                r = r * 0.25
                plsc.store_scatter(sbuf, [jnp.full((L,), row, jnp.int32), lanes],
                                   r, mask=lanes < H)
                macc = jnp.maximum(macc, jnp.where(lanes < H, r, neg))
            return macc

        maxacc = lax.fori_loop(0, G, group_body, maxacc)
        pltpu.sync_copy(sbuf, score.at[pl.ds(base + b * BATCH, BATCH)])
        return maxacc

    start(0, kbuf0, qbuf0, sem0)
    start(1, kbuf1, qbuf1, sem1)

    def pair_body(i, maxacc):
        b0 = 2 * i
        wait(b0, kbuf0, qbuf0, sem0)
        maxacc = compute(b0, kbuf0, qbuf0, maxacc)
        start(b0 + 2, kbuf0, qbuf0, sem0)
        b1 = 2 * i + 1
        wait(b1, kbuf1, qbuf1, sem1)
        maxacc = compute(b1, kbuf1, qbuf1, maxacc)

        @pl.when(i < (NB - 1) // 2 - 1)
        def _():
            start(b1 + 2, kbuf1, qbuf1, sem1)

        return maxacc

    maxacc = lax.fori_loop(0, (NB - 1) // 2, pair_body,
                           jnp.full((L,), -jnp.inf, jnp.float32))
    wait(NB - 1, kbuf0, qbuf0, sem0)
    maxacc = compute(NB - 1, kbuf0, qbuf0, maxacc)
    mbuf[...] = maxacc
    pltpu.sync_copy(mbuf, tmax.at[wid])


def _scores(src, dst, k, q):
    mesh = plsc.VectorSubcoreMesh(core_axis_name="c", subcore_axis_name="s",
                                  num_cores=NC, num_subcores=NS)
    f = pl.kernel(
        _score_body,
        out_type=(jax.ShapeDtypeStruct((E, H), jnp.float32),
                  jax.ShapeDtypeStruct((NW, L), jnp.float32)),
        mesh=mesh,
        compiler_params=pltpu.CompilerParams(needs_layout_passes=False),
        scratch_types=[
            pltpu.VMEM((EPW,), jnp.int32),
            pltpu.VMEM((EPW,), jnp.int32),
            pltpu.VMEM((BATCH, D), jnp.float32),
            pltpu.VMEM((BATCH, D), jnp.float32),
            pltpu.VMEM((BATCH, D), jnp.float32),
            pltpu.VMEM((BATCH, D), jnp.float32),
            pltpu.VMEM((BATCH, H), jnp.float32),
            pltpu.VMEM((L,), jnp.float32),
            pltpu.SemaphoreType.DMA,
            pltpu.SemaphoreType.DMA,
        ],
    )
    return f(src, dst, k, q)


def _agg_body(src, dst, v_hbm, score, tmax, zm, hpart,
              sidx0, didx, vbuf0, msg, sbuf, tmv, epad, h_sp,
              sem0):
    cid = lax.axis_index("c")
    sid = lax.axis_index("s")
    wid = sid * NC + cid
    base = wid * EPW
    lanes = jnp.arange(L, dtype=jnp.int32)

    pltpu.sync_copy(zm, msg)

    def zero_chunk(i, carry):
        c = jnp.minimum(sid * CPT + i, NCHUNK - 1)
        pltpu.sync_copy(msg, h_sp.at[pl.ds(c * CROWS, CROWS)])
        return carry

    lax.fori_loop(0, CPT, zero_chunk, 0)

    pltpu.sync_copy(tmax, tmv)
    mv = jnp.full((L,), -jnp.inf, jnp.float32)
    for i in range(NW):
        mv = jnp.maximum(mv, tmv[i])
    M = jnp.max(mv)

    plsc.subcore_barrier()

    def start(b, sidx, buf, sem):
        pltpu.sync_copy(src.at[pl.ds(base + b * BATCH, BATCH)], sidx)
        pltpu.async_copy(v_hbm.at[sidx], buf, sem)

    def wait(sidx, buf, sem):
        pltpu.make_async_copy(v_hbm.at[sidx], buf, sem).wait()

    def compute(b, vbuf):
        off = base + b * BATCH
        pltpu.sync_copy(dst.at[pl.ds(off, BATCH)], didx)
        pltpu.sync_copy(score.at[pl.ds(off, BATCH)], sbuf)

        def group_body(g, c2):
            rows = g * L + lanes
            for h in range(H):
                hc = jnp.full((L,), h, jnp.int32)
                sv = plsc.load_gather(sbuf, [rows, hc])
                ev = jnp.exp(sv - M)
                plsc.store_scatter(epad, [rows * 9 + h], ev)
            for j in range(L):
                row = g * L + j
                for h in range(H):
                    se = plsc.load_gather(
                        epad, [jnp.full((L,), row * 9 + h, jnp.int32)])
                    c0 = h * HD
                    msg[row, pl.ds(c0, HD)] = vbuf[row, pl.ds(c0, HD)] * se
            return c2

        lax.fori_loop(0, G, group_body, 0)
        pltpu.sync_copy(msg, h_sp.at[didx], add=True)

    def batch_body(b, carry):
        start(b, sidx0, vbuf0, sem0)
        wait(sidx0, vbuf0, sem0)
        compute(b, vbuf0)
        return carry

    lax.fori_loop(0, NB, batch_body, 0)

    plsc.subcore_barrier()

    def out_chunk(i, carry):
        c = jnp.minimum(sid * CPT + i, NCHUNK - 1)
        pltpu.sync_copy(h_sp.at[pl.ds(c * CROWS, CROWS)], vbuf0)
        pltpu.sync_copy(vbuf0, hpart.at[cid, c])
        return carry

    lax.fori_loop(0, CPT, out_chunk, 0)


def _aggregate(src, dst, v, score, tmax):
    mesh = plsc.VectorSubcoreMesh(core_axis_name="c", subcore_axis_name="s",
                                  num_cores=NC, num_subcores=NS)
    zm = jnp.zeros((BATCH, D), jnp.float32)
    f = pl.kernel(
        _agg_body,
        out_type=jax.ShapeDtypeStruct((NC, NCHUNK, CROWS, D), jnp.float32),
        mesh=mesh,
        compiler_params=pltpu.CompilerParams(needs_layout_passes=False),
        scratch_types=[
            pltpu.VMEM((BATCH,), jnp.int32),
            pltpu.VMEM((BATCH,), jnp.int32),
            pltpu.VMEM((BATCH, D), jnp.float32),
            pltpu.VMEM((BATCH, D), jnp.float32),
            pltpu.VMEM((BATCH, H), jnp.float32),
            pltpu.VMEM((NW, L), jnp.float32),
            pltpu.VMEM((BATCH * 9,), jnp.float32),
            pltpu.VMEM_SHARED((N, D), jnp.float32),
            pltpu.SemaphoreType.DMA,
        ],
    )
    return f(src, dst, v, score, tmax, zm)


def _den_body(dst, score, tmax, zd, dpart, didx, sbuf, tmv, dtile):
    cid = lax.axis_index("c")
    sid = lax.axis_index("s")
    wid = sid * NC + cid
    base = wid * EPW
    lanes = jnp.arange(L, dtype=jnp.int32)

    pltpu.sync_copy(zd, dtile)

    pltpu.sync_copy(tmax, tmv)
    mv = jnp.full((L,), -jnp.inf, jnp.float32)
    for i in range(NW):
        mv = jnp.maximum(mv, tmv[i])
    M = jnp.max(mv)

    def batch_body(b, carry):
        off = base + b * DBATCH
        pltpu.sync_copy(dst.at[pl.ds(off, DBATCH)], didx)
        pltpu.sync_copy(score.at[pl.ds(off, DBATCH)], sbuf)

        def group_body(g, c2):
            rows = g * L + lanes
            dd = didx[pl.ds(g * L, L)] * H
            for h in range(H):
                hc = jnp.full((L,), h, jnp.int32)
                sv = plsc.load_gather(sbuf, [rows, hc])
                ev = jnp.exp(sv - M)
                plsc.addupdate_scatter(dtile, [dd + h], ev)
            return c2

        lax.fori_loop(0, DBATCH // L, group_body, 0)
        return carry

    lax.fori_loop(0, EPW // DBATCH, batch_body, 0)
    pltpu.sync_copy(dtile, dpart.at[cid, sid])


def _denominators(dst, score, tmax):
    mesh = plsc.VectorSubcoreMesh(core_axis_name="c", subcore_axis_name="s",
                                  num_cores=NC, num_subcores=NS)
    zd = jnp.zeros((N * H,), jnp.float32)
    f = pl.kernel(
        _den_body,
        out_type=jax.ShapeDtypeStruct((NC, NS, N * H), jnp.float32),
        mesh=mesh,
        compiler_params=pltpu.CompilerParams(needs_layout_passes=False),
        scratch_types=[
            pltpu.VMEM((DBATCH,), jnp.int32),
            pltpu.VMEM((DBATCH, H), jnp.float32),
            pltpu.VMEM((NW, L), jnp.float32),
            pltpu.VMEM((N * H,), jnp.float32),
        ],
    )
    return f(dst, score, tmax, zd)


def _final_body(hp0, hp1, dpp, out):
    hsum = hp0[...] + hp1[...]
    d8 = jnp.sum(dpp[...], axis=0)
    safe = jnp.where(d8 == 0.0, 1.0, d8)
    r = 1.0 / safe
    rowi = lax.broadcasted_iota(jnp.int32, (H, D), 0)
    coli = lax.broadcasted_iota(jnp.int32, (H, D), 1)
    em = jnp.where(rowi == coli // HD, 1.0, 0.0).astype(jnp.float32)
    dex = lax.dot_general(r, em, (((1,), (0,)), ((), ())),
                          preferred_element_type=jnp.float32)
    out[...] = hsum * dex


def _finalize(hp0, hp1, dpp):
    grid = (N // _RB,)
    h_spec = pl.BlockSpec((_RB, D), lambda i: (i, 0))
    d_spec = pl.BlockSpec((NW, _RB, H), lambda i: (0, i, 0))
    return pl.pallas_call(
        _final_body,
        grid=grid,
        in_specs=[h_spec, h_spec, d_spec],
        out_specs=h_spec,
        out_shape=jax.ShapeDtypeStruct((N, D), jnp.float32),
    )(hp0, hp1, dpp)


def kernel(feat, edge_index, W_Q, b_Q, W_K, b_K, W_V, b_V):
    src = edge_index[0]
    dst = edge_index[1]
    q, k, v = _project(feat, W_Q, b_Q, W_K, b_K, W_V, b_V)
    score, tmax = _scores(src, dst, k, q)
    hp = _aggregate(src, dst, v, score, tmax)
    dp = _denominators(dst, score, tmax)
    hp = hp.reshape(NC, N, D)
    dpp = dp.reshape(NW, N, H)
    return _finalize(hp[0], hp[1], dpp)
